# baseline (speedup 1.0000x reference)
import jax
import jax.numpy as jnp
from jax.experimental import pallas as pl
from jax.experimental.pallas import tpu as pltpu

_WIDTH = 2048
_KPAD = 2176
_TAIL = 8
_NE = 64
_BM = 1024


def _router_body(off_ref, x_ref, t_ref, w_ref, b_ref, o_ref):
    wfull = jnp.concatenate(
        [w_ref[...], jnp.zeros((_NE, _KPAD - _WIDTH), jnp.float32)], axis=1
    )
    wp = pltpu.roll(wfull, off_ref[0], axis=1)
    dn = (((1,), (1,)), ((), ()))
    acc = jax.lax.dot_general(
        wp[:, :_WIDTH], x_ref[...], dimension_numbers=dn,
        preferred_element_type=jnp.float32,
    )
    acc += jax.lax.dot_general(
        wp[:, _WIDTH:_WIDTH + _TAIL],
        t_ref[...].reshape(_BM, _TAIL),
        dimension_numbers=dn,
        preferred_element_type=jnp.float32,
    )
    rows = jax.lax.broadcasted_iota(jnp.int32, (_NE, _NE), 0)
    cols = jax.lax.broadcasted_iota(jnp.int32, (_NE, _NE), 1)
    eye = jnp.where(rows == cols, 1.0, 0.0).astype(jnp.float32)
    b_col = jax.lax.dot_general(
        eye, b_ref[...], dimension_numbers=dn,
        preferred_element_type=jnp.float32,
    )
    o_ref[...] = acc + b_col


def kernel(x, A, W, b):
    n = x.shape[0]
    a32 = A.astype(jnp.int32) if hasattr(A, "astype") else jnp.int32(A)
    off = jnp.reshape(a32 - _WIDTH, (1,))
    b2 = b.reshape(1, _NE)
    xt = x.reshape(n, x.shape[1] // _TAIL, 1, _TAIL)

    out_t = pl.pallas_call(
        _router_body,
        grid_spec=pltpu.PrefetchScalarGridSpec(
            num_scalar_prefetch=1,
            grid=(n // _BM,),
            in_specs=[
                pl.BlockSpec((_BM, _WIDTH), lambda m, off_ref: (m, 0)),
                pl.BlockSpec(
                    (_BM, 1, 1, _TAIL),
                    lambda m, off_ref: (m, _WIDTH // _TAIL, 0, 0),
                ),
                pl.BlockSpec((_NE, _WIDTH), lambda m, off_ref: (0, 0)),
                pl.BlockSpec((1, _NE), lambda m, off_ref: (0, 0)),
            ],
            out_specs=pl.BlockSpec((_NE, _BM), lambda m, off_ref: (0, m)),
        ),
        out_shape=jax.ShapeDtypeStruct((_NE, n), jnp.float32),
        compiler_params=pltpu.CompilerParams(
            dimension_semantics=("parallel",),
        ),
    )(off, x, xt, W, b2)
    return out_t.T


# speedup vs baseline: 8.4636x; 8.4636x over previous
import jax
import jax.numpy as jnp
from jax.experimental import pallas as pl
from jax.experimental.pallas import tpu as pltpu

_WIDTH = 2048
_KPAD = 2176
_TAIL = 128
_NE = 64
_BM = 1024


def _router_body(off_ref, x_ref, t_ref, w_ref, b_ref, o_ref):
    wfull = jnp.concatenate(
        [w_ref[...], jnp.zeros((_NE, _KPAD - _WIDTH), jnp.float32)], axis=1
    )
    wp = pltpu.roll(wfull, off_ref[0], axis=1)
    dn = (((1,), (1,)), ((), ()))
    acc = jax.lax.dot_general(
        wp[:, :_WIDTH], x_ref[...], dimension_numbers=dn,
        preferred_element_type=jnp.float32,
    )
    acc += jax.lax.dot_general(
        wp[:, _WIDTH:_WIDTH + _TAIL],
        t_ref[...].reshape(_BM, _TAIL),
        dimension_numbers=dn,
        preferred_element_type=jnp.float32,
    )
    rows = jax.lax.broadcasted_iota(jnp.int32, (_NE, _NE), 0)
    cols = jax.lax.broadcasted_iota(jnp.int32, (_NE, _NE), 1)
    eye = jnp.where(rows == cols, 1.0, 0.0).astype(jnp.float32)
    b_col = jax.lax.dot_general(
        eye, b_ref[...], dimension_numbers=dn,
        preferred_element_type=jnp.float32,
    )
    o_ref[...] = acc + b_col


def kernel(x, A, W, b):
    n = x.shape[0]
    a32 = A.astype(jnp.int32) if hasattr(A, "astype") else jnp.int32(A)
    off = jnp.reshape(a32 - _WIDTH, (1,))
    b2 = b.reshape(1, _NE)
    xt = x.reshape(n, x.shape[1] // _TAIL, 1, _TAIL)

    out_t = pl.pallas_call(
        _router_body,
        grid_spec=pltpu.PrefetchScalarGridSpec(
            num_scalar_prefetch=1,
            grid=(n // _BM,),
            in_specs=[
                pl.BlockSpec((_BM, _WIDTH), lambda m, off_ref: (m, 0)),
                pl.BlockSpec(
                    (_BM, 1, 1, _TAIL),
                    lambda m, off_ref: (m, _WIDTH // _TAIL, 0, 0),
                ),
                pl.BlockSpec((_NE, _WIDTH), lambda m, off_ref: (0, 0)),
                pl.BlockSpec((1, _NE), lambda m, off_ref: (0, 0)),
            ],
            out_specs=pl.BlockSpec((_NE, _BM), lambda m, off_ref: (0, m)),
        ),
        out_shape=jax.ShapeDtypeStruct((_NE, n), jnp.float32),
        compiler_params=pltpu.CompilerParams(
            dimension_semantics=("parallel",),
        ),
    )(off, x, xt, W, b2)
    return out_t.T


# speedup vs baseline: 154.7102x; 18.2796x over previous
import jax
import jax.numpy as jnp
from jax.experimental import pallas as pl
from jax.experimental.pallas import tpu as pltpu

_WIDTH = 2048
_KPAD = 2176
_NE = 64
_BM = 1024


def _router_body(off_ref, x_ref, w_ref, b_ref, o_ref):
    wfull = jnp.concatenate(
        [w_ref[...], jnp.zeros((_NE, _KPAD - _WIDTH), jnp.float32)], axis=1
    )
    wp = pltpu.roll(wfull, off_ref[0], axis=1)
    acc = jax.lax.dot_general(
        wp, x_ref[...],
        dimension_numbers=(((1,), (1,)), ((), ())),
        preferred_element_type=jnp.float32,
    )
    rows = jax.lax.broadcasted_iota(jnp.int32, (_NE, _NE), 0)
    cols = jax.lax.broadcasted_iota(jnp.int32, (_NE, _NE), 1)
    eye = jnp.where(rows == cols, 1.0, 0.0).astype(jnp.float32)
    b_col = jax.lax.dot_general(
        eye, b_ref[...],
        dimension_numbers=(((1,), (1,)), ((), ())),
        preferred_element_type=jnp.float32,
    )
    o_ref[...] = acc + b_col


def kernel(x, A, W, b):
    n = x.shape[0]
    a32 = A.astype(jnp.int32) if hasattr(A, "astype") else jnp.int32(A)
    off = jnp.reshape(a32 - _WIDTH, (1,))
    b2 = b.reshape(1, _NE)

    out_t = pl.pallas_call(
        _router_body,
        grid_spec=pltpu.PrefetchScalarGridSpec(
            num_scalar_prefetch=1,
            grid=(n // _BM,),
            in_specs=[
                pl.BlockSpec((_BM, _KPAD), lambda m, off_ref: (m, 0)),
                pl.BlockSpec((_NE, _WIDTH), lambda m, off_ref: (0, 0)),
                pl.BlockSpec((1, _NE), lambda m, off_ref: (0, 0)),
            ],
            out_specs=pl.BlockSpec((_NE, _BM), lambda m, off_ref: (0, m)),
        ),
        out_shape=jax.ShapeDtypeStruct((_NE, n), jnp.float32),
        compiler_params=pltpu.CompilerParams(
            dimension_semantics=("arbitrary",),
        ),
    )(off, x, W, b2)
    return out_t.T
